# baseline (speedup 1.0000x reference)
import functools

import jax
import jax.numpy as jnp
from jax import lax
from jax.experimental import pallas as pl
from jax.experimental.pallas import tpu as pltpu
from jax.experimental.pallas import tpu_sc as plsc

B = 16384
D = 64
NC = 2
NS = 16
NW = NC * NS
BPW = B // NW

_mesh = plsc.VectorSubcoreMesh(
    core_axis_name="c", subcore_axis_name="s", num_cores=NC, num_subcores=NS
)


@functools.partial(
    pl.kernel,
    out_type=jax.ShapeDtypeStruct((B, D), jnp.bfloat16),
    mesh=_mesh,
    scratch_types=[
        pltpu.VMEM((BPW,), jnp.int32),
        pltpu.VMEM((BPW, D), jnp.bfloat16),
        pltpu.SemaphoreType.DMA,
    ],
    compiler_params=pltpu.CompilerParams(use_tc_tiling_on_sc=False),
)
def _gather(idx_hbm, table_hbm, out_hbm, idx_v, rows_v, sem):
    wid = lax.axis_index("s") * NC + lax.axis_index("c")
    base = wid * BPW
    pltpu.sync_copy(idx_hbm.at[pl.ds(base, BPW)], idx_v)
    pltpu.async_copy(table_hbm.at[idx_v], rows_v, sem).wait()
    pltpu.sync_copy(rows_v, out_hbm.at[pl.ds(base, BPW)])


@jax.jit
def kernel(inputs, weights):
    return _gather(inputs, weights.astype(jnp.bfloat16))


# speedup vs baseline: 1.2823x; 1.2823x over previous
import functools

import jax
import jax.numpy as jnp
from jax import lax
from jax.experimental import pallas as pl
from jax.experimental.pallas import tpu as pltpu
from jax.experimental.pallas import tpu_sc as plsc

B = 16384
D = 64
NC = 2
NS = 16
NW = NC * NS
BPW = B // NW
QCH = 128

_mesh = plsc.VectorSubcoreMesh(
    core_axis_name="c", subcore_axis_name="s", num_cores=NC, num_subcores=NS
)


@functools.partial(
    pl.kernel,
    out_type=jax.ShapeDtypeStruct((B * D,), jnp.float32),
    mesh=_mesh,
    scratch_types=[
        pltpu.VMEM((BPW,), jnp.int32),
        pltpu.VMEM((BPW,), jnp.int32),
        pltpu.VMEM((QCH, 256), jnp.float32),
        pltpu.VMEM((QCH, 256), jnp.float32),
        pltpu.VMEM((BPW * D,), jnp.float32),
        pltpu.SemaphoreType.DMA,
        pltpu.SemaphoreType.DMA,
    ],
)
def _gather_cast(idx_hbm, table_hbm, out_hbm, idx_v, q_v, quads0, quads1, out_v, sem0, sem1):
    wid = lax.axis_index("s") * NC + lax.axis_index("c")
    base = wid * BPW
    pltpu.sync_copy(idx_hbm.at[pl.ds(base, BPW)], idx_v)

    def quads(i, carry):
        q_v[pl.ds(i * 16, 16)] = idx_v[pl.ds(i * 16, 16)] >> 2
        return carry

    lax.fori_loop(0, BPW // 16, quads, 0, unroll=8)
    bufs = (quads0, quads1)
    sems = (sem0, sem1)

    def start(t):
        return pltpu.async_copy(
            table_hbm.at[q_v.at[pl.ds(t * QCH, QCH)]], bufs[t % 2], sems[t % 2]
        )

    def make_extract(quads_v, chunk_base):
        def extract(r, carry):
            jv = idx_v[pl.ds(chunk_base + r * 16, 16)] & 3
            for k in range(16):
                i = r * 16 + k
                j = jv[k]
                o = pl.multiple_of((chunk_base + i) * D, D)
                for h in range(4):
                    cands = [
                        quads_v[i, pl.ds(c * D + h * 16, 16)] for c in range(4)
                    ]
                    v = jnp.where(
                        j == 0,
                        cands[0],
                        jnp.where(
                            j == 1, cands[1], jnp.where(j == 2, cands[2], cands[3])
                        ),
                    )
                    out_v[pl.ds(o + h * 16, 16)] = v
            return carry

        return extract

    nchunks = BPW // QCH
    cps = [start(0), start(1)]
    for t in range(nchunks):
        cps[t].wait()
        lax.fori_loop(0, QCH // 16, make_extract(bufs[t % 2], t * QCH), 0)
        if t + 2 < nchunks:
            cps.append(start(t + 2))
    pltpu.sync_copy(out_v, out_hbm.at[pl.ds(base * D, BPW * D)])


@jax.jit
def kernel(inputs, weights):
    wq = weights.reshape(250000, 256)
    return _gather_cast(inputs, wq).reshape(B, D).astype(jnp.bfloat16)


# speedup vs baseline: 1.3156x; 1.0260x over previous
import functools

import jax
import jax.numpy as jnp
from jax import lax
from jax.experimental import pallas as pl
from jax.experimental.pallas import tpu as pltpu
from jax.experimental.pallas import tpu_sc as plsc

B = 16384
D = 64
NC = 2
NS = 16
NW = NC * NS
BPW = B // NW

_mesh = plsc.VectorSubcoreMesh(
    core_axis_name="c", subcore_axis_name="s", num_cores=NC, num_subcores=NS
)


@functools.partial(
    pl.kernel,
    out_type=jax.ShapeDtypeStruct((B * D,), jnp.bfloat16),
    mesh=_mesh,
    scratch_types=[
        pltpu.VMEM((BPW,), jnp.int32),
        pltpu.VMEM((BPW, D), jnp.float32),
        pltpu.VMEM((BPW * D,), jnp.bfloat16),
        pltpu.SemaphoreType.DMA,
    ],
    compiler_params=pltpu.CompilerParams(use_tc_tiling_on_sc=False),
)
def _gather_cast(idx_hbm, table_hbm, out_hbm, idx_v, rows_v, out_v, sem):
    wid = lax.axis_index("s") * NC + lax.axis_index("c")
    base = wid * BPW
    pltpu.sync_copy(idx_hbm.at[pl.ds(base, BPW)], idx_v)
    pltpu.async_copy(table_hbm.at[idx_v], rows_v, sem).wait()

    def cast_row(r, carry):
        for h in range(4):
            out_v[pl.ds(r * D + h * 16, 16)] = rows_v[r, pl.ds(h * 16, 16)].astype(
                jnp.bfloat16
            )
        return carry

    lax.fori_loop(0, BPW, cast_row, 0, unroll=4)
    pltpu.sync_copy(out_v, out_hbm.at[pl.ds(base * D, BPW * D)])


@jax.jit
def kernel(inputs, weights):
    return _gather_cast(inputs, weights).reshape(B, D)
